# baseline (speedup 1.0000x reference)
import functools

import jax
import jax.numpy as jnp
from jax import lax
from jax.experimental import pallas as pl
from jax.experimental.pallas import tpu as pltpu
from jax.experimental.pallas import tpu_sc as plsc

_E = 64
_ZL = 0.001
_ROWS = 32768
_COLS = 2048
_N = float(_ROWS * _COLS)

_SC_ROWS = 12288
_SC_N = _SC_ROWS * _COLS
_TC_ROWS = _ROWS - _SC_ROWS
_TC_BLK = 512
_TC_GRID = _TC_ROWS // _TC_BLK
_TC_ROW_OFF = _SC_ROWS // _TC_BLK

_NC, _NS, _L = 2, 16, 16
_NW = _NC * _NS
_PER_TILE = _SC_N // _NW
_CHUNK = 32768
_NCHUNKS = _PER_TILE // _CHUNK
_UNROLL = 8


def _sc_reduce_body(x_hbm, out_hbm, buf0, buf1, accv, sem0, sem1):
    wid = lax.axis_index("s") * _NC + lax.axis_index("c")
    base = wid * _PER_TILE
    bufs = (buf0, buf1)
    sems = (sem0, sem1)
    cps = [None, None]
    cps[0] = pltpu.async_copy(x_hbm.at[pl.ds(base, _CHUNK)], buf0, sem0)
    accs = tuple(jnp.zeros((_L,), jnp.float32) for _ in range(_UNROLL))
    for c in range(_NCHUNKS):
        if c + 1 < _NCHUNKS:
            nb = (c + 1) % 2
            cps[nb] = pltpu.async_copy(
                x_hbm.at[pl.ds(base + (c + 1) * _CHUNK, _CHUNK)],
                bufs[nb], sems[nb])
        cps[c % 2].wait()
        buf = bufs[c % 2]

        def body(i, a, buf=buf):
            off = i * (_L * _UNROLL)
            return tuple(
                aj + buf[pl.ds(off + j * _L, _L)] for j, aj in enumerate(a))

        accs = lax.fori_loop(0, _CHUNK // (_L * _UNROLL), body, accs)
    tot = accs[0]
    for a in accs[1:]:
        tot = tot + a
    accv[...] = tot
    pltpu.sync_copy(accv, out_hbm.at[wid])


_sc_reduce = functools.partial(
    pl.kernel,
    mesh=plsc.VectorSubcoreMesh(core_axis_name="c", subcore_axis_name="s"),
    out_type=jax.ShapeDtypeStruct((_NW, _L), jnp.float32),
    scratch_types=[
        pltpu.VMEM((_CHUNK,), jnp.float32),
        pltpu.VMEM((_CHUNK,), jnp.float32),
        pltpu.VMEM((_L,), jnp.float32),
        pltpu.SemaphoreType.DMA,
        pltpu.SemaphoreType.DMA,
    ],
)(_sc_reduce_body)


def _tc_reduce_body(x_ref, out_ref):
    step = pl.program_id(0)

    @pl.when(step == 0)
    def _init():
        out_ref[...] = jnp.zeros_like(out_ref)

    out_ref[...] += jnp.sum(x_ref[...], axis=0, keepdims=True)


def _combine_body(tc_ref, sc_ref, p_ref, mask_ref, probs_ref, loss_ref,
                  topk_ref):
    total = (jnp.sum(tc_ref[...], keepdims=True)
             + jnp.sum(sc_ref[...], keepdims=True))
    x = total / _N
    p = p_ref[...]
    sim = -((p - x) ** 2)
    m = jnp.max(sim, keepdims=True)
    e = jnp.exp(sim - m)
    denom = jnp.sum(e, keepdims=True)
    probs = e / denom

    idx = lax.broadcasted_iota(jnp.int32, (1, _E), 1)
    m1 = jnp.max(probs, keepdims=True)
    i1 = jnp.min(jnp.where(probs == m1, idx, _E), keepdims=True)
    rest = jnp.where(idx == i1, -jnp.inf, probs)
    m2 = jnp.max(rest, keepdims=True)
    i2 = jnp.min(jnp.where(rest == m2, idx, _E), keepdims=True)

    mask_ref[...] = ((idx == i1) | (idx == i2)).astype(jnp.float32)
    probs_ref[...] = probs
    pm = jnp.sum(probs, keepdims=True) / _E
    loss_ref[...] = (pm - 1.0 / _E) ** 2 * _ZL
    k_iota = lax.broadcasted_iota(jnp.int32, (1, 2), 1)
    topk_ref[...] = jnp.where(k_iota == 0, i1, i2)


def kernel(wm_state, prototypes):
    flat = wm_state.reshape(_ROWS * _COLS)
    wm = wm_state.reshape(_ROWS, _COLS)
    pt = prototypes.reshape(1, _E)

    sc_part = _sc_reduce(flat)

    tc_part = pl.pallas_call(
        _tc_reduce_body,
        grid=(_TC_GRID,),
        in_specs=[pl.BlockSpec((_TC_BLK, _COLS),
                               lambda i: (i + _TC_ROW_OFF, 0))],
        out_specs=pl.BlockSpec((1, _COLS), lambda i: (0, 0)),
        out_shape=jax.ShapeDtypeStruct((1, _COLS), jnp.float32),
    )(wm)

    mask, probs, loss, topk = pl.pallas_call(
        _combine_body,
        in_specs=[
            pl.BlockSpec((1, _COLS), lambda: (0, 0)),
            pl.BlockSpec((_NW, _L), lambda: (0, 0)),
            pl.BlockSpec((1, _E), lambda: (0, 0)),
        ],
        out_specs=[
            pl.BlockSpec((1, _E), lambda: (0, 0)),
            pl.BlockSpec((1, _E), lambda: (0, 0)),
            pl.BlockSpec((1, 1), lambda: (0, 0)),
            pl.BlockSpec((1, 2), lambda: (0, 0)),
        ],
        out_shape=[
            jax.ShapeDtypeStruct((1, _E), jnp.float32),
            jax.ShapeDtypeStruct((1, _E), jnp.float32),
            jax.ShapeDtypeStruct((1, 1), jnp.float32),
            jax.ShapeDtypeStruct((1, 2), jnp.int32),
        ],
    )(tc_part, sc_part, pt)
    return (mask.reshape(_E), probs.reshape(_E),
            loss.reshape(()), topk.reshape(2))


# speedup vs baseline: 2.7991x; 2.7991x over previous
import functools

import jax
import jax.numpy as jnp
from jax import lax
from jax.experimental import pallas as pl
from jax.experimental.pallas import tpu as pltpu
from jax.experimental.pallas import tpu_sc as plsc

_E = 64
_ZL = 0.001
_ROWS = 32768
_COLS = 2048
_N = float(_ROWS * _COLS)

_SC_ROWS = 12288
_SC_N = _SC_ROWS * _COLS
_TC_ROWS = _ROWS - _SC_ROWS
_TC_BLK = 512
_TC_GRID = _TC_ROWS // _TC_BLK
_TC_ROW_OFF = _SC_ROWS // _TC_BLK

_NC, _NS, _L = 2, 16, 16
_NW = _NC * _NS
_TILE_ROWS = _SC_ROWS // _NW
_CROWS = 16
_NCHUNKS = _TILE_ROWS // _CROWS
_UNROLL = 8


def _sc_reduce_body(x_hbm, out_hbm, buf0, buf1, accv, sem0, sem1):
    wid = lax.axis_index("s") * _NC + lax.axis_index("c")
    base_row = wid * _TILE_ROWS

    def src(c):
        return x_hbm.at[pl.ds(base_row + c * _CROWS, _CROWS), :]

    pltpu.async_copy(src(0), buf0, sem0)
    pltpu.async_copy(src(1), buf1, sem1)

    def accum_buf(buf, accs):
        for r in range(_CROWS):
            def rbody(g, a, r=r):
                col = g * (_L * _UNROLL)
                return tuple(aj + buf[r, pl.ds(col + j * _L, _L)]
                             for j, aj in enumerate(a))

            accs = lax.fori_loop(0, _COLS // (_L * _UNROLL), rbody, accs)
        return accs

    def body(i, accs):
        c = i * 2
        for b, (buf, sem) in enumerate(((buf0, sem0), (buf1, sem1))):
            pltpu.make_async_copy(src(c + b), buf, sem).wait()
            accs = accum_buf(buf, accs)

            @pl.when(c + b + 2 < _NCHUNKS)
            def _next(buf=buf, sem=sem, c=c, b=b):
                pltpu.async_copy(src(c + b + 2), buf, sem)

        return accs

    accs = tuple(jnp.zeros((_L,), jnp.float32) for _ in range(_UNROLL))
    accs = lax.fori_loop(0, _NCHUNKS // 2, body, accs)
    tot = accs[0]
    for a in accs[1:]:
        tot = tot + a
    accv[...] = tot
    pltpu.sync_copy(accv, out_hbm.at[wid])


_sc_reduce = functools.partial(
    pl.kernel,
    mesh=plsc.VectorSubcoreMesh(core_axis_name="c", subcore_axis_name="s"),
    out_type=jax.ShapeDtypeStruct((_NW, _L), jnp.float32),
    scratch_types=[
        pltpu.VMEM((_CROWS, _COLS), jnp.float32),
        pltpu.VMEM((_CROWS, _COLS), jnp.float32),
        pltpu.VMEM((_L,), jnp.float32),
        pltpu.SemaphoreType.DMA,
        pltpu.SemaphoreType.DMA,
    ],
)(_sc_reduce_body)


def _tc_reduce_body(x_ref, out_ref):
    step = pl.program_id(0)

    @pl.when(step == 0)
    def _init():
        out_ref[...] = jnp.zeros_like(out_ref)

    out_ref[...] += jnp.sum(x_ref[...], axis=0, keepdims=True)


def _combine_body(tc_ref, sc_ref, p_ref, mask_ref, probs_ref, loss_ref,
                  topk_ref):
    total = (jnp.sum(tc_ref[...], keepdims=True)
             + jnp.sum(sc_ref[...], keepdims=True))
    x = total / _N
    p = p_ref[...]
    sim = -((p - x) ** 2)
    m = jnp.max(sim, keepdims=True)
    e = jnp.exp(sim - m)
    denom = jnp.sum(e, keepdims=True)
    probs = e / denom

    idx = lax.broadcasted_iota(jnp.int32, (1, _E), 1)
    m1 = jnp.max(probs, keepdims=True)
    i1 = jnp.min(jnp.where(probs == m1, idx, _E), keepdims=True)
    rest = jnp.where(idx == i1, -jnp.inf, probs)
    m2 = jnp.max(rest, keepdims=True)
    i2 = jnp.min(jnp.where(rest == m2, idx, _E), keepdims=True)

    mask_ref[...] = ((idx == i1) | (idx == i2)).astype(jnp.float32)
    probs_ref[...] = probs
    pm = jnp.sum(probs, keepdims=True) / _E
    loss_ref[...] = (pm - 1.0 / _E) ** 2 * _ZL
    k_iota = lax.broadcasted_iota(jnp.int32, (1, 2), 1)
    topk_ref[...] = jnp.where(k_iota == 0, i1, i2)


def kernel(wm_state, prototypes):
    wm = wm_state.reshape(_ROWS, _COLS)
    pt = prototypes.reshape(1, _E)

    sc_part = _sc_reduce(wm)

    tc_part = pl.pallas_call(
        _tc_reduce_body,
        grid=(_TC_GRID,),
        in_specs=[pl.BlockSpec((_TC_BLK, _COLS),
                               lambda i: (i + _TC_ROW_OFF, 0))],
        out_specs=pl.BlockSpec((1, _COLS), lambda i: (0, 0)),
        out_shape=jax.ShapeDtypeStruct((1, _COLS), jnp.float32),
    )(wm)

    mask, probs, loss, topk = pl.pallas_call(
        _combine_body,
        in_specs=[
            pl.BlockSpec((1, _COLS), lambda: (0, 0)),
            pl.BlockSpec((_NW, _L), lambda: (0, 0)),
            pl.BlockSpec((1, _E), lambda: (0, 0)),
        ],
        out_specs=[
            pl.BlockSpec((1, _E), lambda: (0, 0)),
            pl.BlockSpec((1, _E), lambda: (0, 0)),
            pl.BlockSpec((1, 1), lambda: (0, 0)),
            pl.BlockSpec((1, 2), lambda: (0, 0)),
        ],
        out_shape=[
            jax.ShapeDtypeStruct((1, _E), jnp.float32),
            jax.ShapeDtypeStruct((1, _E), jnp.float32),
            jax.ShapeDtypeStruct((1, 1), jnp.float32),
            jax.ShapeDtypeStruct((1, 2), jnp.int32),
        ],
    )(tc_part, sc_part, pt)
    return (mask.reshape(_E), probs.reshape(_E),
            loss.reshape(()), topk.reshape(2))


# speedup vs baseline: 2.8205x; 1.0076x over previous
import functools

import jax
import jax.numpy as jnp
from jax import lax
from jax.experimental import pallas as pl
from jax.experimental.pallas import tpu as pltpu
from jax.experimental.pallas import tpu_sc as plsc

_E = 64
_ZL = 0.001
_ROWS = 32768
_COLS = 2048
_N = float(_ROWS * _COLS)

_SC_ROWS = 4096
_SC_N = _SC_ROWS * _COLS
_TC_ROWS = _ROWS - _SC_ROWS
_TC_BLK = 512
_TC_GRID = _TC_ROWS // _TC_BLK
_TC_ROW_OFF = _SC_ROWS // _TC_BLK

_NC, _NS, _L = 2, 16, 16
_NW = _NC * _NS
_TILE_ROWS = _SC_ROWS // _NW
_CROWS = 16
_NCHUNKS = _TILE_ROWS // _CROWS
_UNROLL = 8


def _sc_reduce_body(x_hbm, out_hbm, buf0, buf1, accv, sem0, sem1):
    wid = lax.axis_index("s") * _NC + lax.axis_index("c")
    base_row = wid * _TILE_ROWS

    def src(c):
        return x_hbm.at[pl.ds(base_row + c * _CROWS, _CROWS), :]

    pltpu.async_copy(src(0), buf0, sem0)
    pltpu.async_copy(src(1), buf1, sem1)

    def accum_buf(buf, accs):
        for r in range(_CROWS):
            def rbody(g, a, r=r):
                col = g * (_L * _UNROLL)
                return tuple(aj + buf[r, pl.ds(col + j * _L, _L)]
                             for j, aj in enumerate(a))

            accs = lax.fori_loop(0, _COLS // (_L * _UNROLL), rbody, accs)
        return accs

    def body(i, accs):
        c = i * 2
        for b, (buf, sem) in enumerate(((buf0, sem0), (buf1, sem1))):
            pltpu.make_async_copy(src(c + b), buf, sem).wait()
            accs = accum_buf(buf, accs)

            @pl.when(c + b + 2 < _NCHUNKS)
            def _next(buf=buf, sem=sem, c=c, b=b):
                pltpu.async_copy(src(c + b + 2), buf, sem)

        return accs

    accs = tuple(jnp.zeros((_L,), jnp.float32) for _ in range(_UNROLL))
    accs = lax.fori_loop(0, _NCHUNKS // 2, body, accs)
    tot = accs[0]
    for a in accs[1:]:
        tot = tot + a
    accv[...] = tot
    pltpu.sync_copy(accv, out_hbm.at[wid])


_sc_reduce = functools.partial(
    pl.kernel,
    mesh=plsc.VectorSubcoreMesh(core_axis_name="c", subcore_axis_name="s"),
    out_type=jax.ShapeDtypeStruct((_NW, _L), jnp.float32),
    scratch_types=[
        pltpu.VMEM((_CROWS, _COLS), jnp.float32),
        pltpu.VMEM((_CROWS, _COLS), jnp.float32),
        pltpu.VMEM((_L,), jnp.float32),
        pltpu.SemaphoreType.DMA,
        pltpu.SemaphoreType.DMA,
    ],
)(_sc_reduce_body)


def _tc_reduce_body(x_ref, out_ref):
    step = pl.program_id(0)

    @pl.when(step == 0)
    def _init():
        out_ref[...] = jnp.zeros_like(out_ref)

    out_ref[...] += jnp.sum(x_ref[...], axis=0, keepdims=True)


def _combine_body(tc_ref, sc_ref, p_ref, mask_ref, probs_ref, loss_ref,
                  topk_ref):
    total = (jnp.sum(tc_ref[...], keepdims=True)
             + jnp.sum(sc_ref[...], keepdims=True))
    x = total / _N
    p = p_ref[...]
    sim = -((p - x) ** 2)
    m = jnp.max(sim, keepdims=True)
    e = jnp.exp(sim - m)
    denom = jnp.sum(e, keepdims=True)
    probs = e / denom

    idx = lax.broadcasted_iota(jnp.int32, (1, _E), 1)
    m1 = jnp.max(probs, keepdims=True)
    i1 = jnp.min(jnp.where(probs == m1, idx, _E), keepdims=True)
    rest = jnp.where(idx == i1, -jnp.inf, probs)
    m2 = jnp.max(rest, keepdims=True)
    i2 = jnp.min(jnp.where(rest == m2, idx, _E), keepdims=True)

    mask_ref[...] = ((idx == i1) | (idx == i2)).astype(jnp.float32)
    probs_ref[...] = probs
    pm = jnp.sum(probs, keepdims=True) / _E
    loss_ref[...] = (pm - 1.0 / _E) ** 2 * _ZL
    k_iota = lax.broadcasted_iota(jnp.int32, (1, 2), 1)
    topk_ref[...] = jnp.where(k_iota == 0, i1, i2)


def kernel(wm_state, prototypes):
    wm = wm_state.reshape(_ROWS, _COLS)
    pt = prototypes.reshape(1, _E)

    sc_part = _sc_reduce(wm)

    tc_part = pl.pallas_call(
        _tc_reduce_body,
        grid=(_TC_GRID,),
        in_specs=[pl.BlockSpec((_TC_BLK, _COLS),
                               lambda i: (i + _TC_ROW_OFF, 0))],
        out_specs=pl.BlockSpec((1, _COLS), lambda i: (0, 0)),
        out_shape=jax.ShapeDtypeStruct((1, _COLS), jnp.float32),
    )(wm)

    mask, probs, loss, topk = pl.pallas_call(
        _combine_body,
        in_specs=[
            pl.BlockSpec((1, _COLS), lambda: (0, 0)),
            pl.BlockSpec((_NW, _L), lambda: (0, 0)),
            pl.BlockSpec((1, _E), lambda: (0, 0)),
        ],
        out_specs=[
            pl.BlockSpec((1, _E), lambda: (0, 0)),
            pl.BlockSpec((1, _E), lambda: (0, 0)),
            pl.BlockSpec((1, 1), lambda: (0, 0)),
            pl.BlockSpec((1, 2), lambda: (0, 0)),
        ],
        out_shape=[
            jax.ShapeDtypeStruct((1, _E), jnp.float32),
            jax.ShapeDtypeStruct((1, _E), jnp.float32),
            jax.ShapeDtypeStruct((1, 1), jnp.float32),
            jax.ShapeDtypeStruct((1, 2), jnp.int32),
        ],
    )(tc_part, sc_part, pt)
    return (mask.reshape(_E), probs.reshape(_E),
            loss.reshape(()), topk.reshape(2))


# speedup vs baseline: 3.5366x; 1.2539x over previous
import jax
import jax.numpy as jnp
from jax import lax
from jax.experimental import pallas as pl
from jax.experimental.pallas import tpu as pltpu

_E = 64
_ZL = 0.001
_ROWS = 32768
_COLS = 2048
_BLK = 1024
_N = float(_ROWS * _COLS)


def _body(x_ref, p_ref, mask_ref, probs_ref, loss_ref, topk_ref, acc_ref):
    step = pl.program_id(0)

    @pl.when(step == 0)
    def _init():
        acc_ref[...] = jnp.zeros_like(acc_ref)

    acc_ref[...] += jnp.sum(x_ref[...], axis=0, keepdims=True)

    @pl.when(step == pl.num_programs(0) - 1)
    def _finish():
        total = jnp.sum(acc_ref[...], keepdims=True)
        x = total / _N
        p = p_ref[...]
        sim = -((p - x) ** 2)
        m = jnp.max(sim, keepdims=True)
        e = jnp.exp(sim - m)
        denom = jnp.sum(e, keepdims=True)
        probs = e / denom

        idx = lax.broadcasted_iota(jnp.int32, (1, _E), 1)
        m1 = jnp.max(probs, keepdims=True)
        i1 = jnp.min(jnp.where(probs == m1, idx, _E), keepdims=True)
        rest = jnp.where(idx == i1, -jnp.inf, probs)
        m2 = jnp.max(rest, keepdims=True)
        i2 = jnp.min(jnp.where(rest == m2, idx, _E), keepdims=True)

        mask_ref[...] = ((idx == i1) | (idx == i2)).astype(jnp.float32)
        probs_ref[...] = probs
        pm = jnp.sum(probs, keepdims=True) / _E
        loss_ref[...] = (pm - 1.0 / _E) ** 2 * _ZL
        k_iota = lax.broadcasted_iota(jnp.int32, (1, 2), 1)
        topk_ref[...] = jnp.where(k_iota == 0, i1, i2)


def kernel(wm_state, prototypes):
    wm = wm_state.reshape(_ROWS, _COLS)
    pt = prototypes.reshape(1, _E)
    grid = _ROWS // _BLK
    mask, probs, loss, topk = pl.pallas_call(
        _body,
        grid=(grid,),
        in_specs=[
            pl.BlockSpec((_BLK, _COLS), lambda i: (i, 0)),
            pl.BlockSpec((1, _E), lambda i: (0, 0)),
        ],
        out_specs=[
            pl.BlockSpec((1, _E), lambda i: (0, 0)),
            pl.BlockSpec((1, _E), lambda i: (0, 0)),
            pl.BlockSpec((1, 1), lambda i: (0, 0)),
            pl.BlockSpec((1, 2), lambda i: (0, 0)),
        ],
        out_shape=[
            jax.ShapeDtypeStruct((1, _E), jnp.float32),
            jax.ShapeDtypeStruct((1, _E), jnp.float32),
            jax.ShapeDtypeStruct((1, 1), jnp.float32),
            jax.ShapeDtypeStruct((1, 2), jnp.int32),
        ],
        scratch_shapes=[pltpu.VMEM((1, _COLS), jnp.float32)],
    )(wm, pt)
    return (mask.reshape(_E), probs.reshape(_E),
            loss.reshape(()), topk.reshape(2))
